# baseline (speedup 1.0000x reference)
import jax
import jax.numpy as jnp
from jax.experimental import pallas as pl


def _gat_head(x, src, dst, W, a, n):
    z = x @ W.T
    s = z @ a[0, : W.shape[0]]
    t = z @ a[0, W.shape[0]:]
    e = jax.nn.leaky_relu(s[src] + t[dst], negative_slope=0.01)
    emax = jax.ops.segment_max(e, dst, num_segments=n)
    emax = jnp.where(jnp.isfinite(emax), emax, 0.0)
    ex = jnp.exp(e - emax[dst])
    denom = jax.ops.segment_sum(ex, dst, num_segments=n)
    alpha = ex / (denom[dst] + 1e-16)
    return jax.ops.segment_sum(alpha[:, None] * z[src], dst, num_segments=n)


def _elu_kernel(h_ref, o_ref):
    h = h_ref[...]
    o_ref[...] = jnp.where(h > 0, h, jnp.exp(h) - 1.0)


def kernel(x, edge_index, W1, a1, W2, a2, Wp, bp, Wv, bv):
    n = x.shape[0]
    src = edge_index[0]
    dst = edge_index[1]
    heads = [_gat_head(x, src, dst, W1[i], a1[i], n) for i in range(W1.shape[0])]
    h = jnp.concatenate(heads, axis=1)
    h = pl.pallas_call(
        _elu_kernel,
        out_shape=jax.ShapeDtypeStruct(h.shape, jnp.float32),
    )(h)
    h = _gat_head(h, src, dst, W2, a2, n)
    PIo = h @ Wp.T + bp
    mN = jnp.mean(h, axis=0, keepdims=True)
    Vo = mN @ Wv.T + bv
    return (PIo, Vo)


# speedup vs baseline: 40.4814x; 40.4814x over previous
import functools

import jax
import jax.numpy as jnp
from jax import lax
from jax.experimental import pallas as pl
from jax.experimental.pallas import tpu as pltpu
from jax.experimental.pallas import tpu_sc as plsc

N = 10000
E = 320000
D = 128
H = 64
HEADS = 4

NC, NS, L = 2, 16, 16
NW = NC * NS
EPT = E // NW
CH = 80
NCHUNK = EPT // CH
WID = H + L
NPAD = 10240
RPT = NPAD // NS


def _leaky(u):
    return jnp.maximum(u, 0.01 * u)


def _tc1_body(x_ref, w1_ref, a1_ref, z_ref, s_ref, t_ref, b_ref):
    x = x_ref[...]
    z = jax.lax.dot_general(
        x, w1_ref[0], (((1,), (1,)), ((), ())),
        preferred_element_type=jnp.float32,
    )
    z_ref[0] = z
    s = jnp.sum(z * a1_ref[0, 0, :H][None, :], axis=1)
    t = jnp.sum(z * a1_ref[0, 0, H:][None, :], axis=1)
    b = _leaky(jnp.max(s) + t)
    s_ref[0, 0, :] = s
    t_ref[0, 0, :] = t
    b_ref[0, 0, :] = b


def _tc2a_body(p_ref, w2_ref, z2_ref):
    cols = []
    for h in range(HEADS):
        blk = p_ref[2 * h] + p_ref[2 * h + 1]
        hh = blk[:, :H] / (blk[:, H:H + 1] + 1e-30)
        cols.append(jnp.where(hh > 0, hh, jnp.exp(hh) - 1.0))
    hcat = jnp.concatenate(cols, axis=1)
    z2_ref[...] = jax.lax.dot_general(
        hcat, w2_ref[...], (((1,), (1,)), ((), ())),
        preferred_element_type=jnp.float32,
    )


def _tc2b_body(z2_ref, a2_ref, tab2_ref):
    z2 = z2_ref[:N, :]
    s = jnp.sum(z2 * a2_ref[0, :H][None, :], axis=1)
    t = jnp.sum(z2 * a2_ref[0, H:][None, :], axis=1)
    b = _leaky(jnp.max(s) + t)
    tab2_ref[0, :] = s
    tab2_ref[1, :] = t
    tab2_ref[2, :] = b


def _tc3_body(p_ref, wp_ref, bp_ref, wv_ref, bv_ref, pi_ref, v_ref):
    blk = p_ref[0, :N, :] + p_ref[1, :N, :]
    h2 = blk[:, :H] / (blk[:, H:H + 1] + 1e-30)
    bp_s = jnp.sum(bp_ref[...])
    bv_s = jnp.sum(bv_ref[...])
    pi_ref[0, :] = jnp.sum(h2 * wp_ref[...][0][None, :], axis=1) + bp_s
    mn = jnp.mean(h2, axis=0)
    v_ref[0, :] = jnp.full((128,), jnp.sum(mn * wv_ref[...][0]) + bv_s)


def _sc_edge_body(heads, src_hbm, dst_hbm, z_hbm, s_hbm, t_hbm, b_hbm, zer_hbm,
                  out_hbm, src_v, dst_v, s_v, t_v, b_v, w_v, idxc, zbuf,
                  scaled, gsem, acc):
    cid = lax.axis_index("c")
    sid = lax.axis_index("s")
    g = cid * NS + sid
    pltpu.sync_copy(src_hbm.at[g], src_v)
    pltpu.sync_copy(dst_hbm.at[g], dst_v)
    lane = lax.iota(jnp.int32, L)

    for h in range(heads):
        pltpu.sync_copy(s_hbm.at[h, 0], s_v)
        pltpu.sync_copy(t_hbm.at[h, 0], t_v)
        pltpu.sync_copy(b_hbm.at[h, 0], b_v)
        pltpu.sync_copy(zer_hbm, acc.at[pl.ds(sid * RPT, RPT)])
        plsc.subcore_barrier()

        def chunk(cc, carry):
            for v in range(CH // L):
                s16 = src_v[cc, pl.ds(v * L, L)]
                d16 = dst_v[cc, pl.ds(v * L, L)]
                idxc[pl.ds(v * L, L)] = d16
                sg = plsc.load_gather(s_v, [s16])
                tg = plsc.load_gather(t_v, [d16])
                bg = plsc.load_gather(b_v, [d16])
                w_v[pl.ds(v * L, L)] = jnp.exp(_leaky(sg + tg) - bg)
            pltpu.async_copy(z_hbm.at[h].at[src_v.at[cc]], zbuf, gsem).wait()
            for v in range(CH // L):
                w16 = w_v[pl.ds(v * L, L)]
                for j in range(L):
                    r = v * L + j
                    wsp = jnp.full((L,), w16[j])
                    for c in range(H // L):
                        scaled[r, pl.ds(c * L, L)] = (
                            zbuf[r, pl.ds(c * L, L)] * wsp)
                    scaled[r, pl.ds(H, L)] = jnp.where(lane == 0, wsp, 0.0)
            pltpu.sync_copy(scaled, acc.at[idxc], add=True)
            return carry

        lax.fori_loop(0, NCHUNK, chunk, 0)
        plsc.subcore_barrier()
        pltpu.sync_copy(acc.at[pl.ds(sid * RPT, RPT)],
                        out_hbm.at[2 * h + cid, pl.ds(sid * RPT, RPT)])
        plsc.subcore_barrier()


def _sc_edge_call(heads):
    mesh = plsc.VectorSubcoreMesh(core_axis_name="c", subcore_axis_name="s")
    return pl.kernel(
        functools.partial(_sc_edge_body, heads),
        out_type=jax.ShapeDtypeStruct((2 * heads, NPAD, WID), jnp.float32),
        mesh=mesh,
        compiler_params=pltpu.CompilerParams(
            needs_layout_passes=False, use_tc_tiling_on_sc=False),
        scratch_types=[
            pltpu.VMEM((NCHUNK, CH), jnp.int32),
            pltpu.VMEM((NCHUNK, CH), jnp.int32),
            pltpu.VMEM((N,), jnp.float32),
            pltpu.VMEM((N,), jnp.float32),
            pltpu.VMEM((N,), jnp.float32),
            pltpu.VMEM((CH,), jnp.float32),
            pltpu.VMEM((CH,), jnp.int32),
            pltpu.VMEM((CH, H), jnp.float32),
            pltpu.VMEM((CH, WID), jnp.float32),
            pltpu.SemaphoreType.DMA,
            pltpu.VMEM_SHARED((NPAD, WID), jnp.float32),
        ],
    )


def kernel(x, edge_index, W1, a1, W2, a2, Wp, bp, Wv, bv):
    src3 = edge_index[0].reshape(NW, NCHUNK, CH)
    dst3 = edge_index[1].reshape(NW, NCHUNK, CH)
    a1r = a1.reshape(HEADS, 1, 2 * H)
    zeros = jnp.zeros((RPT, WID), jnp.float32)

    z1, s1, t1, b1 = pl.pallas_call(
        _tc1_body,
        grid=(HEADS,),
        in_specs=[
            pl.BlockSpec((N, D), lambda h: (0, 0)),
            pl.BlockSpec((1, H, D), lambda h: (h, 0, 0)),
            pl.BlockSpec((1, 1, 2 * H), lambda h: (h, 0, 0)),
        ],
        out_specs=(
            pl.BlockSpec((1, N, H), lambda h: (h, 0, 0)),
            pl.BlockSpec((1, 1, N), lambda h: (h, 0, 0)),
            pl.BlockSpec((1, 1, N), lambda h: (h, 0, 0)),
            pl.BlockSpec((1, 1, N), lambda h: (h, 0, 0)),
        ),
        out_shape=(
            jax.ShapeDtypeStruct((HEADS, N, H), jnp.float32),
            jax.ShapeDtypeStruct((HEADS, 1, N), jnp.float32),
            jax.ShapeDtypeStruct((HEADS, 1, N), jnp.float32),
            jax.ShapeDtypeStruct((HEADS, 1, N), jnp.float32),
        ),
    )(x, W1, a1r)

    part1 = _sc_edge_call(HEADS)(src3, dst3, z1, s1, t1, b1, zeros)

    RB = 2048
    z2p = pl.pallas_call(
        _tc2a_body,
        grid=(NPAD // RB,),
        in_specs=[
            pl.BlockSpec((2 * HEADS, RB, WID), lambda r: (0, r, 0)),
            pl.BlockSpec((H, HEADS * H), lambda r: (0, 0)),
        ],
        out_specs=pl.BlockSpec((RB, H), lambda r: (r, 0)),
        out_shape=jax.ShapeDtypeStruct((NPAD, H), jnp.float32),
    )(part1, W2)

    tab2 = pl.pallas_call(
        _tc2b_body,
        out_shape=jax.ShapeDtypeStruct((3, N), jnp.float32),
    )(z2p, a2)

    part2 = _sc_edge_call(1)(
        src3, dst3, z2p[None, :N], tab2[0][None, None], tab2[1][None, None],
        tab2[2][None, None], zeros)

    pi_row, vrow = pl.pallas_call(
        _tc3_body,
        out_shape=(
            jax.ShapeDtypeStruct((1, N), jnp.float32),
            jax.ShapeDtypeStruct((1, 128), jnp.float32),
        ),
    )(part2, Wp, bp, Wv, bv)

    return (pi_row[0][:, None], vrow[:, :1])


# speedup vs baseline: 74.1241x; 1.8311x over previous
import functools

import jax
import jax.numpy as jnp
from jax import lax
from jax.experimental import pallas as pl
from jax.experimental.pallas import tpu as pltpu
from jax.experimental.pallas import tpu_sc as plsc

N = 10000
E = 320000
D = 128
H = 64
HEADS = 4

NC, NS, L = 2, 16, 16
NW = NC * NS
EPT = E // NW
CH = 80
NCHUNK = EPT // CH
WID = H + L
NPAD = 10240
RPT = NPAD // NS


def _leaky(u):
    return jnp.maximum(u, 0.01 * u)


def _tc1_body(x_ref, w1_ref, a1_ref, z_ref, s_ref, t_ref, b_ref):
    x = x_ref[...]
    z = jax.lax.dot_general(
        x, w1_ref[0], (((1,), (1,)), ((), ())),
        preferred_element_type=jnp.float32,
    )
    z_ref[0] = z
    s = jnp.sum(z * a1_ref[0, 0, :H][None, :], axis=1)
    t = jnp.sum(z * a1_ref[0, 0, H:][None, :], axis=1)
    b = _leaky(jnp.max(s) + t)
    s_ref[0, 0, :] = s
    t_ref[0, 0, :] = t
    b_ref[0, 0, :] = b


def _tc2a_body(p_ref, w2_ref, z2_ref):
    cols = []
    for h in range(HEADS):
        blk = p_ref[2 * h] + p_ref[2 * h + 1]
        hh = blk[:, :H] / (blk[:, H:H + 1] + 1e-30)
        cols.append(jnp.where(hh > 0, hh, jnp.exp(hh) - 1.0))
    hcat = jnp.concatenate(cols, axis=1)
    z2_ref[...] = jax.lax.dot_general(
        hcat, w2_ref[...], (((1,), (1,)), ((), ())),
        preferred_element_type=jnp.float32,
    )


def _tc2b_body(z2_ref, a2_ref, tab2_ref):
    z2 = z2_ref[:N, :]
    s = jnp.sum(z2 * a2_ref[0, :H][None, :], axis=1)
    t = jnp.sum(z2 * a2_ref[0, H:][None, :], axis=1)
    b = _leaky(jnp.max(s) + t)
    tab2_ref[0, :] = s
    tab2_ref[1, :] = t
    tab2_ref[2, :] = b


def _tc3_body(p_ref, wp_ref, bp_ref, wv_ref, bv_ref, pi_ref, v_ref):
    blk = p_ref[0, :N, :] + p_ref[1, :N, :]
    h2 = blk[:, :H] / (blk[:, H:H + 1] + 1e-30)
    bp_s = jnp.sum(bp_ref[...])
    bv_s = jnp.sum(bv_ref[...])
    pi_ref[0, :] = jnp.sum(h2 * wp_ref[...][0][None, :], axis=1) + bp_s
    mn = jnp.mean(h2, axis=0)
    v_ref[0, :] = jnp.full((128,), jnp.sum(mn * wv_ref[...][0]) + bv_s)


def _sc_edge_body(heads, src_hbm, dst_hbm, z_hbm, s_hbm, t_hbm, b_hbm, zer_hbm,
                  out_hbm, src_v, dst_v, s_v, t_v, b_v, w_v, idxc, zbuf,
                  scaled, gsem, ssem, acc):
    cid = lax.axis_index("c")
    sid = lax.axis_index("s")
    g = cid * NS + sid
    pltpu.sync_copy(src_hbm.at[g], src_v)
    pltpu.sync_copy(dst_hbm.at[g], dst_v)
    lane = lax.iota(jnp.int32, L)

    for h in range(heads):
        pltpu.sync_copy(s_hbm.at[h, 0], s_v)
        pltpu.sync_copy(t_hbm.at[h, 0], t_v)
        pltpu.sync_copy(b_hbm.at[h, 0], b_v)
        pltpu.sync_copy(zer_hbm, acc.at[pl.ds(sid * RPT, RPT)])
        plsc.subcore_barrier()

        for p in range(2):
            pltpu.async_copy(z_hbm.at[h].at[src_v.at[p]], zbuf[p], gsem[p])

        def stage(cc, p):
            for v in range(CH // L):
                s16 = src_v[cc, pl.ds(v * L, L)]
                d16 = dst_v[cc, pl.ds(v * L, L)]
                idxc[p][pl.ds(v * L, L)] = d16
                sg = plsc.load_gather(s_v, [s16])
                tg = plsc.load_gather(t_v, [d16])
                bg = plsc.load_gather(b_v, [d16])
                w_v[p][pl.ds(v * L, L)] = jnp.exp(_leaky(sg + tg) - bg)
            pltpu.make_async_copy(
                z_hbm.at[h].at[src_v.at[cc]], zbuf[p], gsem[p]).wait()
            for v in range(CH // L):
                w16 = w_v[p][pl.ds(v * L, L)]
                for j in range(L):
                    r = v * L + j
                    wsp = jnp.full((L,), w16[j])
                    for c in range(H // L):
                        scaled[p][r, pl.ds(c * L, L)] = (
                            zbuf[p][r, pl.ds(c * L, L)] * wsp)
                    scaled[p][r, pl.ds(H, L)] = jnp.where(lane == 0, wsp, 0.0)

            @pl.when(cc + 2 < NCHUNK)
            def _():
                pltpu.async_copy(
                    z_hbm.at[h].at[src_v.at[cc + 2]], zbuf[p], gsem[p])

            pltpu.async_copy(scaled[p], acc.at[idxc[p]], ssem[p], add=True)

        def pair(i, carry):
            cc = i * 2
            for p in range(2):
                @pl.when(cc + p >= 2)
                def _():
                    pltpu.make_async_copy(
                        scaled[p], acc.at[idxc[p]], ssem[p]).wait()
                stage(cc + p, p)
            return carry

        lax.fori_loop(0, NCHUNK // 2, pair, 0)
        pltpu.make_async_copy(scaled[0], acc.at[idxc[0]], ssem[0]).wait()
        stage(NCHUNK - 1, 0)
        pltpu.make_async_copy(scaled[0], acc.at[idxc[0]], ssem[0]).wait()
        pltpu.make_async_copy(scaled[1], acc.at[idxc[1]], ssem[1]).wait()
        plsc.subcore_barrier()
        pltpu.sync_copy(acc.at[pl.ds(sid * RPT, RPT)],
                        out_hbm.at[2 * h + cid, pl.ds(sid * RPT, RPT)])
        plsc.subcore_barrier()


def _sc_edge_call(heads):
    mesh = plsc.VectorSubcoreMesh(core_axis_name="c", subcore_axis_name="s")
    return pl.kernel(
        functools.partial(_sc_edge_body, heads),
        out_type=jax.ShapeDtypeStruct((2 * heads, NPAD, WID), jnp.float32),
        mesh=mesh,
        compiler_params=pltpu.CompilerParams(
            needs_layout_passes=False, use_tc_tiling_on_sc=False),
        scratch_types=[
            pltpu.VMEM((NCHUNK, CH), jnp.int32),
            pltpu.VMEM((NCHUNK, CH), jnp.int32),
            pltpu.VMEM((N,), jnp.float32),
            pltpu.VMEM((N,), jnp.float32),
            pltpu.VMEM((N,), jnp.float32),
            [pltpu.VMEM((CH,), jnp.float32) for _ in range(2)],
            [pltpu.VMEM((CH,), jnp.int32) for _ in range(2)],
            [pltpu.VMEM((CH, H), jnp.float32) for _ in range(2)],
            [pltpu.VMEM((CH, WID), jnp.float32) for _ in range(2)],
            [pltpu.SemaphoreType.DMA for _ in range(2)],
            [pltpu.SemaphoreType.DMA for _ in range(2)],
            pltpu.VMEM_SHARED((NPAD, WID), jnp.float32),
        ],
    )


def kernel(x, edge_index, W1, a1, W2, a2, Wp, bp, Wv, bv):
    src3 = edge_index[0].reshape(NW, NCHUNK, CH)
    dst3 = edge_index[1].reshape(NW, NCHUNK, CH)
    a1r = a1.reshape(HEADS, 1, 2 * H)
    zeros = jnp.zeros((RPT, WID), jnp.float32)

    z1, s1, t1, b1 = pl.pallas_call(
        _tc1_body,
        grid=(HEADS,),
        in_specs=[
            pl.BlockSpec((N, D), lambda h: (0, 0)),
            pl.BlockSpec((1, H, D), lambda h: (h, 0, 0)),
            pl.BlockSpec((1, 1, 2 * H), lambda h: (h, 0, 0)),
        ],
        out_specs=(
            pl.BlockSpec((1, N, H), lambda h: (h, 0, 0)),
            pl.BlockSpec((1, 1, N), lambda h: (h, 0, 0)),
            pl.BlockSpec((1, 1, N), lambda h: (h, 0, 0)),
            pl.BlockSpec((1, 1, N), lambda h: (h, 0, 0)),
        ),
        out_shape=(
            jax.ShapeDtypeStruct((HEADS, N, H), jnp.float32),
            jax.ShapeDtypeStruct((HEADS, 1, N), jnp.float32),
            jax.ShapeDtypeStruct((HEADS, 1, N), jnp.float32),
            jax.ShapeDtypeStruct((HEADS, 1, N), jnp.float32),
        ),
    )(x, W1, a1r)

    part1 = _sc_edge_call(HEADS)(src3, dst3, z1, s1, t1, b1, zeros)

    RB = 2048
    z2p = pl.pallas_call(
        _tc2a_body,
        grid=(NPAD // RB,),
        in_specs=[
            pl.BlockSpec((2 * HEADS, RB, WID), lambda r: (0, r, 0)),
            pl.BlockSpec((H, HEADS * H), lambda r: (0, 0)),
        ],
        out_specs=pl.BlockSpec((RB, H), lambda r: (r, 0)),
        out_shape=jax.ShapeDtypeStruct((NPAD, H), jnp.float32),
    )(part1, W2)

    tab2 = pl.pallas_call(
        _tc2b_body,
        out_shape=jax.ShapeDtypeStruct((3, N), jnp.float32),
    )(z2p, a2)

    part2 = _sc_edge_call(1)(
        src3, dst3, z2p[None, :N], tab2[0][None, None], tab2[1][None, None],
        tab2[2][None, None], zeros)

    pi_row, vrow = pl.pallas_call(
        _tc3_body,
        out_shape=(
            jax.ShapeDtypeStruct((1, N), jnp.float32),
            jax.ShapeDtypeStruct((1, 128), jnp.float32),
        ),
    )(part2, Wp, bp, Wv, bv)

    return (pi_row[0][:, None], vrow[:, :1])
